# initial kernel scaffold (unmeasured)
import jax
import jax.numpy as jnp
from jax import lax
from jax.experimental import pallas as pl
from jax.experimental.pallas import tpu as pltpu

N_DEV = 8
M_PER = 1024
D = 1024
F_PER = 4096
SUB = 4
R = M_PER // SUB


def kernel(x, W1, W2):
    x = x.astype(jnp.bfloat16)
    W1 = W1.astype(jnp.bfloat16)
    W2 = W2.astype(jnp.bfloat16)

    def body(x_ref, w1_ref, w2_ref, out_ref, xg_ref, part_ref,
             ag_send, ag_recv, rs_send, rs_recv, reg_sems):
        me = lax.axis_index("i")
        left = lax.rem(me + (N_DEV - 1), N_DEV)
        right = lax.rem(me + 1, N_DEV)

        bar = pltpu.get_barrier_semaphore()
        for nbr in (left, right):
            pl.semaphore_signal(bar, inc=1, device_id=(nbr,),
                                device_id_type=pl.DeviceIdType.MESH)
        pl.semaphore_wait(bar, 2)

        for h in range(N_DEV - 1):
            src = x_ref if h == 0 else xg_ref.at[h]
            rdma = pltpu.make_async_remote_copy(
                src_ref=src,
                dst_ref=xg_ref.at[h + 1],
                send_sem=ag_send.at[h],
                recv_sem=ag_recv.at[h],
                device_id=(right,),
                device_id_type=pl.DeviceIdType.MESH,
            )
            rdma.start()
            rdma.wait()

        for s in range(N_DEV):
            xs = x_ref if s == 0 else xg_ref.at[s]
            for j in range(SUB):
                xa = xs[pl.ds(j * R, R), :]
                h = jnp.dot(xa, w1_ref[:, :],
                            preferred_element_type=jnp.float32)
                h = h * jax.nn.sigmoid(h)
                p = jnp.dot(h.astype(jnp.bfloat16), w2_ref[:, :],
                            preferred_element_type=jnp.float32)
                part_ref[s, pl.ds(j * R, R), :] = p.astype(jnp.bfloat16)

        for nbr in (left, right):
            pl.semaphore_signal(reg_sems.at[0], inc=1, device_id=(nbr,),
                                device_id_type=pl.DeviceIdType.MESH)
        pl.semaphore_wait(reg_sems.at[0], 2)

        for t in range(N_DEV - 1):
            if t > 0:
                part_ref[t + 1, :, :] = (
                    part_ref[t + 1, :, :] + xg_ref[t - 1, :, :]
                )
            rdma = pltpu.make_async_remote_copy(
                src_ref=part_ref.at[t + 1],
                dst_ref=xg_ref.at[t],
                send_sem=rs_send.at[t],
                recv_sem=rs_recv.at[t],
                device_id=(right,),
                device_id_type=pl.DeviceIdType.MESH,
            )
            rdma.start()
            rdma.wait()

        out_ref[:, :] = (part_ref[0, :, :].astype(jnp.float32)
                         + xg_ref[N_DEV - 2, :, :].astype(jnp.float32))

        for nbr in (left, right):
            pl.semaphore_signal(reg_sems.at[1], inc=1, device_id=(nbr,),
                                device_id_type=pl.DeviceIdType.MESH)
        pl.semaphore_wait(reg_sems.at[1], 2)

    return pl.pallas_call(
        body,
        out_shape=jax.ShapeDtypeStruct((M_PER, D), jnp.float32),
        in_specs=[
            pl.BlockSpec(memory_space=pltpu.VMEM),
            pl.BlockSpec(memory_space=pltpu.VMEM),
            pl.BlockSpec(memory_space=pltpu.VMEM),
        ],
        out_specs=pl.BlockSpec(memory_space=pltpu.VMEM),
        scratch_shapes=[
            pltpu.VMEM((N_DEV, M_PER, D), jnp.bfloat16),
            pltpu.VMEM((N_DEV, M_PER, D), jnp.bfloat16),
            pltpu.SemaphoreType.DMA((N_DEV - 1,)),
            pltpu.SemaphoreType.DMA((N_DEV - 1,)),
            pltpu.SemaphoreType.DMA((N_DEV - 1,)),
            pltpu.SemaphoreType.DMA((N_DEV - 1,)),
            pltpu.SemaphoreType.REGULAR((2,)),
        ],
        compiler_params=pltpu.CompilerParams(collective_id=0),
    )(x, W1, W2)


# baseline (device time: 532849 ns/iter reference)
import jax
import jax.numpy as jnp
from jax import lax
from jax.experimental import pallas as pl
from jax.experimental.pallas import tpu as pltpu

N_DEV = 8
M_PER = 1024
D = 1024
F_PER = 4096
SUB = 8
R = M_PER // SUB


def kernel(x, W1, W2):
    x = x.astype(jnp.bfloat16)
    W1 = W1.astype(jnp.bfloat16)
    W2 = W2.astype(jnp.bfloat16)

    def body(x_ref, w1_ref, w2_ref, out_ref, xg_ref, racc_ref,
             ag_send, ag_recv, rs_send, rs_recv, reg_sems):
        me = lax.axis_index("i")
        left = lax.rem(me + (N_DEV - 1), N_DEV)
        right = lax.rem(me + 1, N_DEV)

        bar = pltpu.get_barrier_semaphore()
        for nbr in (left, right):
            pl.semaphore_signal(bar, inc=1, device_id=(nbr,),
                                device_id_type=pl.DeviceIdType.MESH)
        pl.semaphore_wait(bar, 2)

        xg_ref[0, :, :] = x_ref[:, :]
        for h in range(N_DEV - 1):
            rdma = pltpu.make_async_remote_copy(
                src_ref=xg_ref.at[h],
                dst_ref=xg_ref.at[h + 1],
                send_sem=ag_send.at[h],
                recv_sem=ag_recv.at[h],
                device_id=(right,),
                device_id_type=pl.DeviceIdType.MESH,
            )
            rdma.start()
            rdma.wait()

        for s in range(N_DEV):
            xs = xg_ref.at[s]

            def tile(j, _, xs=xs):
                xa = xs[pl.ds(j * R, R), :]
                h = jnp.dot(xa, w1_ref[:, :],
                            preferred_element_type=jnp.float32)
                h = h * jax.nn.sigmoid(h)
                p = jnp.dot(h.astype(jnp.bfloat16), w2_ref[:, :],
                            preferred_element_type=jnp.float32)
                xs[pl.ds(j * R, R), :] = p.astype(jnp.bfloat16)
                return _

            lax.fori_loop(0, SUB, tile, None)

        for t in range(N_DEV - 1):
            if t > 0:
                xg_ref[t + 1, :, :] = (
                    xg_ref[t + 1, :, :] + racc_ref[(t - 1) % 2, :, :]
                )
                pl.semaphore_signal(reg_sems.at[(t - 1) % 2], inc=1,
                                    device_id=(left,),
                                    device_id_type=pl.DeviceIdType.MESH)
            if t >= 2:
                pl.semaphore_wait(reg_sems.at[t % 2], 1)
            rdma = pltpu.make_async_remote_copy(
                src_ref=xg_ref.at[t + 1],
                dst_ref=racc_ref.at[t % 2],
                send_sem=rs_send.at[t],
                recv_sem=rs_recv.at[t],
                device_id=(right,),
                device_id_type=pl.DeviceIdType.MESH,
            )
            rdma.start()
            rdma.wait()

        out_ref[:, :] = (xg_ref[0, :, :].astype(jnp.float32)
                         + racc_ref[0, :, :].astype(jnp.float32))

        pl.semaphore_wait(reg_sems.at[1], 1)

        for nbr in (left, right):
            pl.semaphore_signal(reg_sems.at[2], inc=1, device_id=(nbr,),
                                device_id_type=pl.DeviceIdType.MESH)
        pl.semaphore_wait(reg_sems.at[2], 2)

    return pl.pallas_call(
        body,
        out_shape=jax.ShapeDtypeStruct((M_PER, D), jnp.float32),
        in_specs=[
            pl.BlockSpec(memory_space=pltpu.VMEM),
            pl.BlockSpec(memory_space=pltpu.VMEM),
            pl.BlockSpec(memory_space=pltpu.VMEM),
        ],
        out_specs=pl.BlockSpec(memory_space=pltpu.VMEM),
        scratch_shapes=[
            pltpu.VMEM((N_DEV, M_PER, D), jnp.bfloat16),
            pltpu.VMEM((2, M_PER, D), jnp.bfloat16),
            pltpu.SemaphoreType.DMA((N_DEV - 1,)),
            pltpu.SemaphoreType.DMA((N_DEV - 1,)),
            pltpu.SemaphoreType.DMA((N_DEV - 1,)),
            pltpu.SemaphoreType.DMA((N_DEV - 1,)),
            pltpu.SemaphoreType.REGULAR((3,)),
        ],
        compiler_params=pltpu.CompilerParams(
            collective_id=0,
            vmem_limit_bytes=40 * 1024 * 1024,
        ),
    )(x, W1, W2)
